# baseline (speedup 1.0000x reference)
import functools

import jax
import jax.numpy as jnp
from jax import lax
from jax.experimental import pallas as pl
from jax.experimental.pallas import tpu as pltpu
from jax.experimental.pallas import tpu_sc as plsc

B = 128
N = 32768
K_RANK = 1639
NWORKERS = 32
ROWS_PER_W = B // NWORKERS
CHUNKS = N // 16
NBINS = 2048
INT_MIN_I32 = jnp.int32(-(2**31))
INT_MAX_I32 = jnp.int32(2**31 - 1)


def _desc_key(u):
    return u ^ (INT_MIN_I32 | ~(u >> 31))


def _inv_desc_key(kd):
    return jnp.where(kd >= 0, kd ^ INT_MIN_I32, ~kd)


_mesh = plsc.VectorSubcoreMesh(core_axis_name="c", subcore_axis_name="s")


@functools.partial(
    pl.kernel,
    out_type=jax.ShapeDtypeStruct((B, N), jnp.float32),
    mesh=_mesh,
    compiler_params=pltpu.CompilerParams(needs_layout_passes=False),
    scratch_types=[
        pltpu.VMEM((N,), jnp.float32),
        pltpu.VMEM((N,), jnp.int32),
        pltpu.VMEM((NBINS,), jnp.int32),
    ],
)
def _kwta_sc(x_hbm, out_hbm, row_f, row_kd, hist):
    wid = lax.axis_index("s") * 2 + lax.axis_index("c")
    zeros16 = jnp.zeros((16,), jnp.int32)
    ones16 = jnp.ones((16,), jnp.int32)

    def zero_hist(nbins):
        @plsc.parallel_loop(0, nbins // 16, 1, unroll=8)
        def _(c):
            hist[pl.ds(c * 16, 16)] = zeros16

    def scan_hist(nbins, r):
        def body(c, carry):
            csum, nlt, before = carry
            v = hist[pl.ds(c * 16, 16)]
            cum = csum + plsc.cumsum(v)
            lt = cum < r
            nlt = nlt + jnp.sum(lt.astype(jnp.int32))
            before = jnp.maximum(before, jnp.max(jnp.where(lt, cum, 0)))
            csum = jnp.max(cum)
            return csum, nlt, before

        z = jnp.int32(0)
        _, nlt, before = lax.fori_loop(0, nbins // 16, body, (z, z, z))
        return nlt, before

    def do_row(i, _):
        row = wid * ROWS_PER_W + i
        pltpu.sync_copy(x_hbm.at[row], row_f)

        zero_hist(NBINS)

        @plsc.parallel_loop(0, CHUNKS, 1, unroll=4)
        def _(c):
            xv = row_f[pl.ds(c * 16, 16)]
            u = lax.bitcast_convert_type(xv, jnp.int32)
            kd = _desc_key(u)
            row_kd[pl.ds(c * 16, 16)] = kd
            ku = kd ^ INT_MIN_I32
            bins = lax.shift_right_logical(ku, 21)
            cnt, lm = plsc.scan_count(bins)
            plsc.addupdate_scatter(hist, [bins], cnt, mask=lm)

        r1 = jnp.int32(K_RANK)
        b1, before1 = scan_hist(NBINS, r1)
        r2 = r1 - before1

        zero_hist(NBINS)

        @plsc.parallel_loop(0, CHUNKS, 1, unroll=4)
        def _(c):
            kd = row_kd[pl.ds(c * 16, 16)]
            ku = kd ^ INT_MIN_I32
            m = lax.shift_right_logical(ku, 21) == b1
            bins = lax.shift_right_logical(ku, 10) & 0x7FF
            cnt, lm = plsc.scan_count(bins, mask=m)
            plsc.addupdate_scatter(hist, [bins], cnt, mask=lm & m)

        b2, before2 = scan_hist(NBINS, r2)
        r3 = r2 - before2
        p2 = (b1 << 11) | b2

        zero_hist(1024)

        @plsc.parallel_loop(0, CHUNKS, 1, unroll=4)
        def _(c):
            kd = row_kd[pl.ds(c * 16, 16)]
            ku = kd ^ INT_MIN_I32
            m = lax.shift_right_logical(ku, 10) == p2
            bins = ku & 0x3FF
            cnt, lm = plsc.scan_count(bins, mask=m)
            plsc.addupdate_scatter(hist, [bins], cnt, mask=lm & m)

        b3, _before3 = scan_hist(1024, r3)
        k1_kd = (((p2 << 10) | b3) ^ INT_MIN_I32).astype(jnp.int32)

        maxs16 = jnp.full((16,), INT_MAX_I32, jnp.int32)

        @plsc.parallel_loop(0, CHUNKS, 1, unroll=4, carry=(zeros16, maxs16))
        def p4(c, carry):
            cnt, mn = carry
            kd = row_kd[pl.ds(c * 16, 16)]
            le = kd <= k1_kd
            cnt = cnt + le.astype(jnp.int32)
            mn = jnp.minimum(mn, jnp.where(le, INT_MAX_I32, kd))
            return cnt, mn

        cnt, mn = p4
        c_le = jnp.sum(cnt)
        k2_kd = jnp.where(c_le >= K_RANK + 1, k1_kd, jnp.min(mn))

        k1v = jnp.full((16,), k1_kd, jnp.int32)
        k2v = jnp.full((16,), k2_kd, jnp.int32)
        va = lax.bitcast_convert_type(_inv_desc_key(k1v), jnp.float32)
        vb = lax.bitcast_convert_type(_inv_desc_key(k2v), jnp.float32)
        t = (va + vb) * jnp.float32(0.5)
        t = jnp.where(t == 0.0, jnp.float32(0.0), t)
        t_kd = _desc_key(lax.bitcast_convert_type(t, jnp.int32))

        @plsc.parallel_loop(0, CHUNKS, 1, unroll=4)
        def _(c):
            kd = row_kd[pl.ds(c * 16, 16)]
            row_f[pl.ds(c * 16, 16)] = jnp.where(
                kd < t_kd, jnp.float32(1.0), jnp.float32(0.0)
            )

        pltpu.sync_copy(row_f, out_hbm.at[row])
        return 0

    lax.fori_loop(0, ROWS_PER_W, do_row, 0)


def kernel(x):
    return _kwta_sc(x)


# speedup vs baseline: 1.2331x; 1.2331x over previous
import functools

import jax
import jax.numpy as jnp
from jax import lax
from jax.experimental import pallas as pl
from jax.experimental.pallas import tpu as pltpu
from jax.experimental.pallas import tpu_sc as plsc

B = 128
N = 32768
K_RANK = 1639
NWORKERS = 32
ROWS_PER_W = B // NWORKERS
CHUNKS = N // 16
NBINS = 2048
INT_MIN_I32 = jnp.int32(-(2**31))
INT_MAX_I32 = jnp.int32(2**31 - 1)


def _desc_key(u):
    return u ^ (INT_MIN_I32 | ~(u >> 31))


def _inv_desc_key(kd):
    return jnp.where(kd >= 0, kd ^ INT_MIN_I32, ~kd)


_mesh = plsc.VectorSubcoreMesh(core_axis_name="c", subcore_axis_name="s")


@functools.partial(
    pl.kernel,
    out_type=jax.ShapeDtypeStruct((B, N), jnp.float32),
    mesh=_mesh,
    compiler_params=pltpu.CompilerParams(needs_layout_passes=False),
    scratch_types=[
        pltpu.VMEM((N,), jnp.float32),
        pltpu.VMEM((N,), jnp.int32),
        pltpu.VMEM((NBINS,), jnp.int32),
        pltpu.SMEM((NBINS // 16,), jnp.int32),
    ],
)
def _kwta_sc(x_hbm, out_hbm, row_f, row_kd, hist, sums):
    wid = lax.axis_index("s") * 2 + lax.axis_index("c")
    zeros16 = jnp.zeros((16,), jnp.int32)
    ones16 = jnp.ones((16,), jnp.int32)

    def zero_hist(nbins):
        @plsc.parallel_loop(0, nbins // 16, 1, unroll=8)
        def _(c):
            hist[pl.ds(c * 16, 16)] = zeros16

    def scan_hist(nbins, r):
        nch = nbins // 16

        @plsc.parallel_loop(0, nch, 1, unroll=8)
        def _(c):
            sums[c] = jnp.sum(hist[pl.ds(c * 16, 16)])

        def body(c, carry):
            csum, nlt, before = carry
            cum = csum + sums[c]
            lt = cum < r
            nlt = nlt + lt.astype(jnp.int32)
            before = jnp.where(lt, cum, before)
            return cum, nlt, before

        z = jnp.int32(0)
        _, chunk, beforec = lax.fori_loop(0, nch, body, (z, z, z), unroll=8)
        rr = r - beforec
        v = hist[pl.ds(chunk * 16, 16)]
        cum = plsc.cumsum(v)
        lt = cum < rr
        nlt = jnp.sum(lt.astype(jnp.int32))
        before_in = jnp.max(jnp.where(lt, cum, 0))
        return chunk * 16 + nlt, beforec + before_in

    def do_row(i, _):
        row = wid * ROWS_PER_W + i
        pltpu.sync_copy(x_hbm.at[row], row_f)

        zero_hist(NBINS)

        @plsc.parallel_loop(0, CHUNKS, 1, unroll=4)
        def _(c):
            xv = row_f[pl.ds(c * 16, 16)]
            u = lax.bitcast_convert_type(xv, jnp.int32)
            kd = _desc_key(u)
            row_kd[pl.ds(c * 16, 16)] = kd
            ku = kd ^ INT_MIN_I32
            bins = lax.shift_right_logical(ku, 21)
            plsc.addupdate_scatter(hist, [bins], ones16)

        r1 = jnp.int32(K_RANK)
        b1, before1 = scan_hist(NBINS, r1)
        r2 = r1 - before1

        zero_hist(NBINS)

        @plsc.parallel_loop(0, CHUNKS, 1, unroll=4)
        def _(c):
            kd = row_kd[pl.ds(c * 16, 16)]
            ku = kd ^ INT_MIN_I32
            m = lax.shift_right_logical(ku, 21) == b1
            bins = lax.shift_right_logical(ku, 10) & 0x7FF
            plsc.addupdate_scatter(hist, [bins], ones16, mask=m)

        b2, before2 = scan_hist(NBINS, r2)
        r3 = r2 - before2
        p2 = (b1 << 11) | b2

        zero_hist(1024)

        @plsc.parallel_loop(0, CHUNKS, 1, unroll=4)
        def _(c):
            kd = row_kd[pl.ds(c * 16, 16)]
            ku = kd ^ INT_MIN_I32
            m = lax.shift_right_logical(ku, 10) == p2
            bins = ku & 0x3FF
            plsc.addupdate_scatter(hist, [bins], ones16, mask=m)

        b3, _before3 = scan_hist(1024, r3)
        k1_kd = (((p2 << 10) | b3) ^ INT_MIN_I32).astype(jnp.int32)

        maxs16 = jnp.full((16,), INT_MAX_I32, jnp.int32)

        @plsc.parallel_loop(0, CHUNKS, 1, unroll=4, carry=(zeros16, maxs16))
        def p4(c, carry):
            cnt, mn = carry
            kd = row_kd[pl.ds(c * 16, 16)]
            le = kd <= k1_kd
            cnt = cnt + le.astype(jnp.int32)
            mn = jnp.minimum(mn, jnp.where(le, INT_MAX_I32, kd))
            return cnt, mn

        cnt, mn = p4
        c_le = jnp.sum(cnt)
        k2_kd = jnp.where(c_le >= K_RANK + 1, k1_kd, jnp.min(mn))

        k1v = jnp.full((16,), k1_kd, jnp.int32)
        k2v = jnp.full((16,), k2_kd, jnp.int32)
        va = lax.bitcast_convert_type(_inv_desc_key(k1v), jnp.float32)
        vb = lax.bitcast_convert_type(_inv_desc_key(k2v), jnp.float32)
        t = (va + vb) * jnp.float32(0.5)
        t = jnp.where(t == 0.0, jnp.float32(0.0), t)
        t_kd = _desc_key(lax.bitcast_convert_type(t, jnp.int32))

        @plsc.parallel_loop(0, CHUNKS, 1, unroll=4)
        def _(c):
            kd = row_kd[pl.ds(c * 16, 16)]
            row_f[pl.ds(c * 16, 16)] = jnp.where(
                kd < t_kd, jnp.float32(1.0), jnp.float32(0.0)
            )

        pltpu.sync_copy(row_f, out_hbm.at[row])
        return 0

    lax.fori_loop(0, ROWS_PER_W, do_row, 0)


def kernel(x):
    return _kwta_sc(x)


# speedup vs baseline: 1.3203x; 1.0707x over previous
import functools

import jax
import jax.numpy as jnp
from jax import lax
from jax.experimental import pallas as pl
from jax.experimental.pallas import tpu as pltpu
from jax.experimental.pallas import tpu_sc as plsc

B = 128
N = 32768
K_RANK = 1639
NWORKERS = 32
ROWS_PER_W = B // NWORKERS
CHUNKS = N // 16
NBINS = 2048
INT_MIN_I32 = jnp.int32(-(2**31))
INT_MAX_I32 = jnp.int32(2**31 - 1)


def _desc_key(u):
    return u ^ (INT_MIN_I32 | ~(u >> 31))


def _inv_desc_key(kd):
    return jnp.where(kd >= 0, kd ^ INT_MIN_I32, ~kd)


_mesh = plsc.VectorSubcoreMesh(core_axis_name="c", subcore_axis_name="s")


@functools.partial(
    pl.kernel,
    out_type=jax.ShapeDtypeStruct((B, N), jnp.float32),
    mesh=_mesh,
    compiler_params=pltpu.CompilerParams(needs_layout_passes=False),
    scratch_types=[
        pltpu.VMEM((N,), jnp.float32),
        pltpu.VMEM((N,), jnp.float32),
        pltpu.VMEM((N,), jnp.int32),
        pltpu.VMEM((NBINS,), jnp.int32),
        pltpu.SMEM((NBINS // 16,), jnp.int32),
        pltpu.SemaphoreType.DMA,
        pltpu.SemaphoreType.DMA,
        pltpu.SemaphoreType.DMA,
        pltpu.SemaphoreType.DMA,
    ],
)
def _kwta_sc(
    x_hbm, out_hbm, row_a, row_b, row_kd, hist, sums, in_a, in_b, out_a, out_b
):
    wid = lax.axis_index("s") * 2 + lax.axis_index("c")
    zeros16 = jnp.zeros((16,), jnp.int32)
    ones16 = jnp.ones((16,), jnp.int32)

    def zero_hist(nbins):
        @plsc.parallel_loop(0, nbins // 16, 1, unroll=8)
        def _(c):
            hist[pl.ds(c * 16, 16)] = zeros16

    def scan_hist(nbins, r):
        nch = nbins // 16

        @plsc.parallel_loop(0, nch, 1, unroll=8)
        def _(c):
            sums[c] = jnp.sum(hist[pl.ds(c * 16, 16)])

        def body(c, carry):
            csum, nlt, before = carry
            cum = csum + sums[c]
            lt = cum < r
            nlt = nlt + lt.astype(jnp.int32)
            before = jnp.where(lt, cum, before)
            return cum, nlt, before

        z = jnp.int32(0)
        _, chunk, beforec = lax.fori_loop(0, nch, body, (z, z, z), unroll=8)
        rr = r - beforec
        v = hist[pl.ds(chunk * 16, 16)]
        cum = plsc.cumsum(v)
        lt = cum < rr
        nlt = jnp.sum(lt.astype(jnp.int32))
        before_in = jnp.max(jnp.where(lt, cum, 0))
        return chunk * 16 + nlt, beforec + before_in

    base = wid * ROWS_PER_W
    bufs = [(row_a, in_a, out_a), (row_b, in_b, out_b)]
    in_handles = [pltpu.async_copy(x_hbm.at[base], row_a, in_a), None]
    out_handles = [None, None]

    for i in range(ROWS_PER_W):
        row_f, in_sem, out_sem = bufs[i % 2]
        in_handles[i % 2].wait()

        zero_hist(NBINS)

        @plsc.parallel_loop(0, CHUNKS, 1, unroll=4)
        def _(c, row_f=row_f):
            xv = row_f[pl.ds(c * 16, 16)]
            u = lax.bitcast_convert_type(xv, jnp.int32)
            kd = _desc_key(u)
            row_kd[pl.ds(c * 16, 16)] = kd
            ku = kd ^ INT_MIN_I32
            bins = lax.shift_right_logical(ku, 21)
            plsc.addupdate_scatter(hist, [bins], ones16)

        if i + 1 < ROWS_PER_W:
            nxt_f, nxt_in, _ = bufs[(i + 1) % 2]
            if out_handles[(i + 1) % 2] is not None:
                out_handles[(i + 1) % 2].wait()
                out_handles[(i + 1) % 2] = None
            in_handles[(i + 1) % 2] = pltpu.async_copy(
                x_hbm.at[base + i + 1], nxt_f, nxt_in
            )

        r1 = jnp.int32(K_RANK)
        b1, before1 = scan_hist(NBINS, r1)
        r2 = r1 - before1

        zero_hist(NBINS)

        @plsc.parallel_loop(0, CHUNKS, 1, unroll=4)
        def _(c):
            kd = row_kd[pl.ds(c * 16, 16)]
            ku = kd ^ INT_MIN_I32
            m = lax.shift_right_logical(ku, 21) == b1
            bins = lax.shift_right_logical(ku, 10) & 0x7FF
            plsc.addupdate_scatter(hist, [bins], ones16, mask=m)

        b2, before2 = scan_hist(NBINS, r2)
        r3 = r2 - before2
        p2 = (b1 << 11) | b2

        zero_hist(1024)

        @plsc.parallel_loop(0, CHUNKS, 1, unroll=4)
        def _(c):
            kd = row_kd[pl.ds(c * 16, 16)]
            ku = kd ^ INT_MIN_I32
            m = lax.shift_right_logical(ku, 10) == p2
            bins = ku & 0x3FF
            plsc.addupdate_scatter(hist, [bins], ones16, mask=m)

        b3, _before3 = scan_hist(1024, r3)
        k1_kd = (((p2 << 10) | b3) ^ INT_MIN_I32).astype(jnp.int32)

        maxs16 = jnp.full((16,), INT_MAX_I32, jnp.int32)

        @plsc.parallel_loop(0, CHUNKS, 1, unroll=4, carry=(zeros16, maxs16))
        def p4(c, carry):
            cnt, mn = carry
            kd = row_kd[pl.ds(c * 16, 16)]
            le = kd <= k1_kd
            cnt = cnt + le.astype(jnp.int32)
            mn = jnp.minimum(mn, jnp.where(le, INT_MAX_I32, kd))
            return cnt, mn

        cnt, mn = p4
        c_le = jnp.sum(cnt)
        k2_kd = jnp.where(c_le >= K_RANK + 1, k1_kd, jnp.min(mn))

        k1v = jnp.full((16,), k1_kd, jnp.int32)
        k2v = jnp.full((16,), k2_kd, jnp.int32)
        va = lax.bitcast_convert_type(_inv_desc_key(k1v), jnp.float32)
        vb = lax.bitcast_convert_type(_inv_desc_key(k2v), jnp.float32)
        t = (va + vb) * jnp.float32(0.5)
        t = jnp.where(t == 0.0, jnp.float32(0.0), t)
        t_kd = _desc_key(lax.bitcast_convert_type(t, jnp.int32))

        @plsc.parallel_loop(0, CHUNKS, 1, unroll=4)
        def _(c, row_f=row_f, t_kd=t_kd):
            kd = row_kd[pl.ds(c * 16, 16)]
            row_f[pl.ds(c * 16, 16)] = jnp.where(
                kd < t_kd, jnp.float32(1.0), jnp.float32(0.0)
            )

        out_handles[i % 2] = pltpu.async_copy(row_f, out_hbm.at[base + i], out_sem)

    for h in out_handles:
        if h is not None:
            h.wait()


def kernel(x):
    return _kwta_sc(x)


# speedup vs baseline: 1.3366x; 1.0124x over previous
import functools

import jax
import jax.numpy as jnp
from jax import lax
from jax.experimental import pallas as pl
from jax.experimental.pallas import tpu as pltpu
from jax.experimental.pallas import tpu_sc as plsc

B = 128
N = 32768
K_RANK = 1639
NWORKERS = 32
ROWS_PER_W = B // NWORKERS
CHUNKS = N // 16
NBINS = 2048
INT_MIN_I32 = jnp.int32(-(2**31))
INT_MAX_I32 = jnp.int32(2**31 - 1)


def _desc_key(u):
    return u ^ (INT_MIN_I32 | ~(u >> 31))


def _inv_desc_key(kd):
    return jnp.where(kd >= 0, kd ^ INT_MIN_I32, ~kd)


_mesh = plsc.VectorSubcoreMesh(core_axis_name="c", subcore_axis_name="s")


@functools.partial(
    pl.kernel,
    out_type=jax.ShapeDtypeStruct((B, N), jnp.float32),
    mesh=_mesh,
    compiler_params=pltpu.CompilerParams(needs_layout_passes=False),
    scratch_types=[
        pltpu.VMEM((N,), jnp.float32),
        pltpu.VMEM((N,), jnp.float32),
        pltpu.VMEM((N,), jnp.int32),
        pltpu.VMEM((NBINS,), jnp.int32),
        pltpu.SMEM((NBINS // 16,), jnp.int32),
        pltpu.SemaphoreType.DMA,
        pltpu.SemaphoreType.DMA,
        pltpu.SemaphoreType.DMA,
        pltpu.SemaphoreType.DMA,
    ],
)
def _kwta_sc(
    x_hbm, out_hbm, row_a, row_b, row_kd, hist, sums, in_a, in_b, out_a, out_b
):
    wid = lax.axis_index("s") * 2 + lax.axis_index("c")
    zeros16 = jnp.zeros((16,), jnp.int32)
    ones16 = jnp.ones((16,), jnp.int32)

    def zero_hist(nbins):
        @plsc.parallel_loop(0, nbins // 16, 1, unroll=8)
        def _(c):
            hist[pl.ds(c * 16, 16)] = zeros16

    def scan_hist(nbins, r):
        nch = nbins // 16

        @plsc.parallel_loop(0, nch, 1, unroll=8)
        def _(c):
            sums[c] = jnp.sum(hist[pl.ds(c * 16, 16)])

        def body(c, carry):
            csum, nlt, before = carry
            cum = csum + sums[c]
            lt = cum < r
            nlt = nlt + lt.astype(jnp.int32)
            before = jnp.where(lt, cum, before)
            return cum, nlt, before

        z = jnp.int32(0)
        _, chunk, beforec = lax.fori_loop(0, nch, body, (z, z, z), unroll=8)
        rr = r - beforec
        v = hist[pl.ds(chunk * 16, 16)]
        cum = plsc.cumsum(v)
        lt = cum < rr
        nlt = jnp.sum(lt.astype(jnp.int32))
        before_in = jnp.max(jnp.where(lt, cum, 0))
        return chunk * 16 + nlt, beforec + before_in

    def scan_hist2(nbins, r):
        nch = nbins // 16

        @plsc.parallel_loop(0, nch, 1, unroll=8)
        def _(c):
            sums[c] = jnp.sum(hist[pl.ds(c * 16, 16)])

        rn = r + 1

        def body(c, carry):
            csum, nlt, before, nlt2, before2 = carry
            cum = csum + sums[c]
            lt = cum < r
            nlt = nlt + lt.astype(jnp.int32)
            before = jnp.where(lt, cum, before)
            lt2 = cum < rn
            nlt2 = nlt2 + lt2.astype(jnp.int32)
            before2 = jnp.where(lt2, cum, before2)
            return cum, nlt, before, nlt2, before2

        z = jnp.int32(0)
        total, chunk, beforec, chunk2, beforec2 = lax.fori_loop(
            0, nch, body, (z, z, z, z, z), unroll=8
        )
        rr = r - beforec
        v = hist[pl.ds(chunk * 16, 16)]
        cum = plsc.cumsum(v)
        lt = cum < rr
        nlt = jnp.sum(lt.astype(jnp.int32))
        before_in = jnp.max(jnp.where(lt, cum, 0))
        bin_a = chunk * 16 + nlt
        before_a = beforec + before_in
        chunk2 = jnp.minimum(chunk2, nch - 1)
        rr2 = rn - beforec2
        v2 = hist[pl.ds(chunk2 * 16, 16)]
        cum2 = plsc.cumsum(v2)
        nlt2f = jnp.sum((cum2 < rr2).astype(jnp.int32))
        bin_b = chunk2 * 16 + nlt2f
        return bin_a, before_a, bin_b, total

    base = wid * ROWS_PER_W
    bufs = [(row_a, in_a, out_a), (row_b, in_b, out_b)]
    in_handles = [pltpu.async_copy(x_hbm.at[base], row_a, in_a), None]
    out_handles = [None, None]

    for i in range(ROWS_PER_W):
        row_f, in_sem, out_sem = bufs[i % 2]
        in_handles[i % 2].wait()

        zero_hist(NBINS)

        @plsc.parallel_loop(0, CHUNKS, 1, unroll=4)
        def _(c, row_f=row_f):
            xv = row_f[pl.ds(c * 16, 16)]
            u = lax.bitcast_convert_type(xv, jnp.int32)
            kd = _desc_key(u)
            row_kd[pl.ds(c * 16, 16)] = kd
            ku = kd ^ INT_MIN_I32
            bins = lax.shift_right_logical(ku, 21)
            plsc.addupdate_scatter(hist, [bins], ones16)

        if i + 1 < ROWS_PER_W:
            nxt_f, nxt_in, _ = bufs[(i + 1) % 2]
            if out_handles[(i + 1) % 2] is not None:
                out_handles[(i + 1) % 2].wait()
                out_handles[(i + 1) % 2] = None
            in_handles[(i + 1) % 2] = pltpu.async_copy(
                x_hbm.at[base + i + 1], nxt_f, nxt_in
            )

        r1 = jnp.int32(K_RANK)
        b1, before1 = scan_hist(NBINS, r1)
        r2 = r1 - before1

        zero_hist(NBINS)

        @plsc.parallel_loop(0, CHUNKS, 1, unroll=4)
        def _(c):
            kd = row_kd[pl.ds(c * 16, 16)]
            ku = kd ^ INT_MIN_I32
            m = lax.shift_right_logical(ku, 21) == b1
            bins = lax.shift_right_logical(ku, 10) & 0x7FF
            plsc.addupdate_scatter(hist, [bins], ones16, mask=m)

        b2, before2 = scan_hist(NBINS, r2)
        r3 = r2 - before2
        p2 = (b1 << 11) | b2

        zero_hist(1024)

        @plsc.parallel_loop(0, CHUNKS, 1, unroll=4)
        def _(c):
            kd = row_kd[pl.ds(c * 16, 16)]
            ku = kd ^ INT_MIN_I32
            m = lax.shift_right_logical(ku, 10) == p2
            bins = ku & 0x3FF
            plsc.addupdate_scatter(hist, [bins], ones16, mask=m)

        b3, _before3, b3n, total3 = scan_hist2(1024, r3)
        k1_kd = (((p2 << 10) | b3) ^ INT_MIN_I32).astype(jnp.int32)

        def k2_fast():
            return (((p2 << 10) | b3n) ^ INT_MIN_I32).astype(jnp.int32)

        def k2_slow():
            maxs16 = jnp.full((16,), INT_MAX_I32, jnp.int32)

            @plsc.parallel_loop(0, CHUNKS, 1, unroll=4, carry=maxs16)
            def p4(c, mn):
                kd = row_kd[pl.ds(c * 16, 16)]
                return jnp.minimum(mn, jnp.where(kd <= k1_kd, INT_MAX_I32, kd))

            return jnp.min(p4)

        k2_kd = lax.cond(total3 >= r3 + 1, k2_fast, k2_slow)

        k1v = jnp.full((16,), k1_kd, jnp.int32)
        k2v = jnp.full((16,), k2_kd, jnp.int32)
        va = lax.bitcast_convert_type(_inv_desc_key(k1v), jnp.float32)
        vb = lax.bitcast_convert_type(_inv_desc_key(k2v), jnp.float32)
        t = (va + vb) * jnp.float32(0.5)
        t = jnp.where(t == 0.0, jnp.float32(0.0), t)
        t_kd = _desc_key(lax.bitcast_convert_type(t, jnp.int32))

        @plsc.parallel_loop(0, CHUNKS, 1, unroll=4)
        def _(c, row_f=row_f, t_kd=t_kd):
            kd = row_kd[pl.ds(c * 16, 16)]
            row_f[pl.ds(c * 16, 16)] = jnp.where(
                kd < t_kd, jnp.float32(1.0), jnp.float32(0.0)
            )

        out_handles[i % 2] = pltpu.async_copy(row_f, out_hbm.at[base + i], out_sem)

    for h in out_handles:
        if h is not None:
            h.wait()


def kernel(x):
    return _kwta_sc(x)


# speedup vs baseline: 1.3518x; 1.0113x over previous
import functools

import jax
import jax.numpy as jnp
from jax import lax
from jax.experimental import pallas as pl
from jax.experimental.pallas import tpu as pltpu
from jax.experimental.pallas import tpu_sc as plsc

B = 128
N = 32768
K_RANK = 1639
NWORKERS = 32
ROWS_PER_W = B // NWORKERS
CHUNKS = N // 16
NBINS = 2048
INT_MIN_I32 = jnp.int32(-(2**31))
INT_MAX_I32 = jnp.int32(2**31 - 1)


def _desc_key(u):
    return u ^ (INT_MIN_I32 | ~(u >> 31))


def _inv_desc_key(kd):
    return jnp.where(kd >= 0, kd ^ INT_MIN_I32, ~kd)


_mesh = plsc.VectorSubcoreMesh(core_axis_name="c", subcore_axis_name="s")


@functools.partial(
    pl.kernel,
    out_type=jax.ShapeDtypeStruct((B, N), jnp.float32),
    mesh=_mesh,
    compiler_params=pltpu.CompilerParams(needs_layout_passes=False),
    scratch_types=[
        pltpu.VMEM((N,), jnp.float32),
        pltpu.VMEM((N,), jnp.float32),
        pltpu.VMEM((N,), jnp.int32),
        pltpu.VMEM((NBINS,), jnp.int32),
        pltpu.SMEM((NBINS // 16,), jnp.int32),
        pltpu.SemaphoreType.DMA,
        pltpu.SemaphoreType.DMA,
        pltpu.SemaphoreType.DMA,
        pltpu.SemaphoreType.DMA,
    ],
)
def _kwta_sc(
    x_hbm, out_hbm, row_a, row_b, row_kd, hist, sums, in_a, in_b, out_a, out_b
):
    wid = lax.axis_index("s") * 2 + lax.axis_index("c")
    zeros16 = jnp.zeros((16,), jnp.int32)
    ones16 = jnp.ones((16,), jnp.int32)

    def zero_hist(nbins):
        @plsc.parallel_loop(0, nbins // 16, 1, unroll=8)
        def _(c):
            hist[pl.ds(c * 16, 16)] = zeros16

    def scan_hist(base, nbins, r):
        nch = nbins // 16

        @plsc.parallel_loop(0, nch, 1, unroll=8)
        def _(c):
            sums[c] = jnp.sum(hist[pl.ds(base + c * 16, 16)])

        def body(c, carry):
            csum, nlt, before = carry
            cum = csum + sums[c]
            lt = cum < r
            nlt = nlt + lt.astype(jnp.int32)
            before = jnp.where(lt, cum, before)
            return cum, nlt, before

        z = jnp.int32(0)
        _, chunk, beforec = lax.fori_loop(0, nch, body, (z, z, z), unroll=8)
        rr = r - beforec
        v = hist[pl.ds(base + chunk * 16, 16)]
        cum = plsc.cumsum(v)
        lt = cum < rr
        nlt = jnp.sum(lt.astype(jnp.int32))
        before_in = jnp.max(jnp.where(lt, cum, 0))
        return chunk * 16 + nlt, beforec + before_in

    def scan_hist2(base, nbins, r):
        nch = nbins // 16

        @plsc.parallel_loop(0, nch, 1, unroll=8)
        def _(c):
            sums[c] = jnp.sum(hist[pl.ds(base + c * 16, 16)])

        rn = r + 1

        def body(c, carry):
            csum, nlt, before, nlt2, before2 = carry
            cum = csum + sums[c]
            lt = cum < r
            nlt = nlt + lt.astype(jnp.int32)
            before = jnp.where(lt, cum, before)
            lt2 = cum < rn
            nlt2 = nlt2 + lt2.astype(jnp.int32)
            before2 = jnp.where(lt2, cum, before2)
            return cum, nlt, before, nlt2, before2

        z = jnp.int32(0)
        total, chunk, beforec, chunk2, beforec2 = lax.fori_loop(
            0, nch, body, (z, z, z, z, z), unroll=8
        )
        rr = r - beforec
        v = hist[pl.ds(base + chunk * 16, 16)]
        cum = plsc.cumsum(v)
        lt = cum < rr
        nlt = jnp.sum(lt.astype(jnp.int32))
        before_in = jnp.max(jnp.where(lt, cum, 0))
        bin_a = chunk * 16 + nlt
        before_a = beforec + before_in
        chunk2 = jnp.minimum(chunk2, nch - 1)
        rr2 = rn - beforec2
        v2 = hist[pl.ds(base + chunk2 * 16, 16)]
        cum2 = plsc.cumsum(v2)
        nlt2f = jnp.sum((cum2 < rr2).astype(jnp.int32))
        bin_b = chunk2 * 16 + nlt2f
        return bin_a, before_a, bin_b, total

    base = wid * ROWS_PER_W
    bufs = [(row_a, in_a, out_a), (row_b, in_b, out_b)]
    in_handles = [pltpu.async_copy(x_hbm.at[base], row_a, in_a), None]
    out_handles = [None, None]

    for i in range(ROWS_PER_W):
        row_f, in_sem, out_sem = bufs[i % 2]
        in_handles[i % 2].wait()

        with jax.named_scope("zero1"):
            zero_hist(NBINS)

        with jax.named_scope("pass1"):

            @plsc.parallel_loop(0, CHUNKS, 1, unroll=4)
            def _(c, row_f=row_f):
                xv = row_f[pl.ds(c * 16, 16)]
                u = lax.bitcast_convert_type(xv, jnp.int32)
                kd = _desc_key(u)
                row_kd[pl.ds(c * 16, 16)] = kd
                ku = kd ^ INT_MIN_I32
                bins = lax.shift_right_logical(ku, 21)
                plsc.addupdate_scatter(hist, [bins], ones16)

        if i + 1 < ROWS_PER_W:
            nxt_f, nxt_in, _ = bufs[(i + 1) % 2]
            if out_handles[(i + 1) % 2] is not None:
                out_handles[(i + 1) % 2].wait()
                out_handles[(i + 1) % 2] = None
            in_handles[(i + 1) % 2] = pltpu.async_copy(
                x_hbm.at[base + i + 1], nxt_f, nxt_in
            )

        with jax.named_scope("scan1"):
            r1 = jnp.int32(K_RANK)
            b1, before1, b1n, _tot1 = scan_hist2(0, NBINS, r1)
            r2 = r1 - before1
            d1 = b1n != b1

        with jax.named_scope("pass2"):
            zero_hist(NBINS)

            @plsc.parallel_loop(0, CHUNKS, 1, unroll=4)
            def _(c):
                kd = row_kd[pl.ds(c * 16, 16)]
                ku = kd ^ INT_MIN_I32
                m = lax.shift_right_logical(ku, 21) == b1
                bins = lax.shift_right_logical(ku, 10) & 0x7FF
                plsc.addupdate_scatter(hist, [bins], ones16, mask=m)

        with jax.named_scope("scan2"):
            b2, before2, b2n, _tot2 = scan_hist2(0, NBINS, r2)
            r3 = r2 - before2
            p2 = (b1 << 11) | b2
            p2n = (b1 << 11) | b2n
            same3 = p2n == p2

        with jax.named_scope("pass3"):
            zero_hist(NBINS)

            @plsc.parallel_loop(0, CHUNKS, 1, unroll=4)
            def _(c):
                kd = row_kd[pl.ds(c * 16, 16)]
                ku = kd ^ INT_MIN_I32
                pfx = lax.shift_right_logical(ku, 10)
                m1 = pfx == p2
                m2 = (pfx == p2n) & jnp.logical_not(same3)
                bins = (ku & 0x3FF) + jnp.where(m2, 1024, 0)
                plsc.addupdate_scatter(hist, [bins], ones16, mask=m1 | m2)

        with jax.named_scope("scan3"):
            b3, _before3, b3n_same, _tot3 = scan_hist2(0, 1024, r3)
            b3n_diff, _bd = scan_hist(1024, 1024, jnp.int32(1))
            k1_kd = (((p2 << 10) | b3) ^ INT_MIN_I32).astype(jnp.int32)
            k2_fast_kd = (
                jnp.where(same3, (p2 << 10) | b3n_same, (p2n << 10) | b3n_diff)
                ^ INT_MIN_I32
            ).astype(jnp.int32)

        with jax.named_scope("k2cond"):

            def k2_slow(k1_kd=k1_kd):
                maxs16 = jnp.full((16,), INT_MAX_I32, jnp.int32)

                @plsc.parallel_loop(0, CHUNKS, 1, unroll=4, carry=maxs16)
                def p4(c, mn):
                    kd = row_kd[pl.ds(c * 16, 16)]
                    return jnp.minimum(
                        mn, jnp.where(kd <= k1_kd, INT_MAX_I32, kd)
                    )

                return jnp.min(p4)

            k2_kd = lax.cond(d1, k2_slow, lambda k2=k2_fast_kd: k2)

        k1v = jnp.full((16,), k1_kd, jnp.int32)
        k2v = jnp.full((16,), k2_kd, jnp.int32)
        va = lax.bitcast_convert_type(_inv_desc_key(k1v), jnp.float32)
        vb = lax.bitcast_convert_type(_inv_desc_key(k2v), jnp.float32)
        t = (va + vb) * jnp.float32(0.5)
        t = jnp.where(t == 0.0, jnp.float32(0.0), t)
        t_kd = _desc_key(lax.bitcast_convert_type(t, jnp.int32))

        with jax.named_scope("maskp"):

            @plsc.parallel_loop(0, CHUNKS, 1, unroll=4)
            def _(c, row_f=row_f, t_kd=t_kd):
                kd = row_kd[pl.ds(c * 16, 16)]
                row_f[pl.ds(c * 16, 16)] = jnp.where(
                    kd < t_kd, jnp.float32(1.0), jnp.float32(0.0)
                )

        out_handles[i % 2] = pltpu.async_copy(
            row_f, out_hbm.at[base + i], out_sem
        )

    for h in out_handles:
        if h is not None:
            h.wait()


def kernel(x):
    return _kwta_sc(x)


# speedup vs baseline: 1.3764x; 1.0182x over previous
import functools

import jax
import jax.numpy as jnp
from jax import lax
from jax.experimental import pallas as pl
from jax.experimental.pallas import tpu as pltpu
from jax.experimental.pallas import tpu_sc as plsc

B = 128
N = 32768
K_RANK = 1639
NWORKERS = 32
ROWS_PER_W = B // NWORKERS
CHUNKS = N // 16
NBINS = 2048
INT_MIN_I32 = jnp.int32(-(2**31))
INT_MAX_I32 = jnp.int32(2**31 - 1)


def _desc_key(u):
    return u ^ (INT_MIN_I32 | ~(u >> 31))


def _inv_desc_key(kd):
    return jnp.where(kd >= 0, kd ^ INT_MIN_I32, ~kd)


_mesh = plsc.VectorSubcoreMesh(core_axis_name="c", subcore_axis_name="s")


@functools.partial(
    pl.kernel,
    out_type=jax.ShapeDtypeStruct((B, N), jnp.float32),
    mesh=_mesh,
    compiler_params=pltpu.CompilerParams(needs_layout_passes=False),
    scratch_types=[
        pltpu.VMEM((N,), jnp.float32),
        pltpu.VMEM((N,), jnp.float32),
        pltpu.VMEM((N,), jnp.int32),
        pltpu.VMEM((NBINS,), jnp.int32),
        pltpu.SMEM((NBINS // 16,), jnp.int32),
        pltpu.SemaphoreType.DMA,
        pltpu.SemaphoreType.DMA,
        pltpu.SemaphoreType.DMA,
        pltpu.SemaphoreType.DMA,
    ],
)
def _kwta_sc(
    x_hbm, out_hbm, row_a, row_b, row_kd, hist, sums, in_a, in_b, out_a, out_b
):
    wid = lax.axis_index("s") * 2 + lax.axis_index("c")
    zeros16 = jnp.zeros((16,), jnp.int32)
    ones16 = jnp.ones((16,), jnp.int32)

    def zero_hist(nbins):
        @plsc.parallel_loop(0, nbins // 16, 1, unroll=8)
        def _(c):
            hist[pl.ds(c * 16, 16)] = zeros16

    def scan_hist(base, nbins, r):
        nch = nbins // 16

        @plsc.parallel_loop(0, nch, 1, unroll=8)
        def _(c):
            sums[c] = jnp.sum(hist[pl.ds(base + c * 16, 16)])

        def body(c, carry):
            csum, nlt, before = carry
            cum = csum + sums[c]
            lt = cum < r
            nlt = nlt + lt.astype(jnp.int32)
            before = jnp.where(lt, cum, before)
            return cum, nlt, before

        z = jnp.int32(0)
        _, chunk, beforec = lax.fori_loop(0, nch, body, (z, z, z), unroll=8)
        rr = r - beforec
        v = hist[pl.ds(base + chunk * 16, 16)]
        cum = plsc.cumsum(v)
        lt = cum < rr
        nlt = jnp.sum(lt.astype(jnp.int32))
        before_in = jnp.max(jnp.where(lt, cum, 0))
        return chunk * 16 + nlt, beforec + before_in

    def scan_hist2(base, nbins, r, remap=False):
        nch = nbins // 16

        @plsc.parallel_loop(0, nch, 1, unroll=8)
        def _(c):
            sums[c] = jnp.sum(hist[pl.ds(base + c * 16, 16)])

        rn = r + 1

        def body(c, carry):
            csum, nlt, before, nlt2, before2 = carry
            rc = (c + nch // 2) & (nch - 1) if remap else c
            cum = csum + sums[rc]
            lt = cum < r
            nlt = nlt + lt.astype(jnp.int32)
            before = jnp.where(lt, cum, before)
            lt2 = cum < rn
            nlt2 = nlt2 + lt2.astype(jnp.int32)
            before2 = jnp.where(lt2, cum, before2)
            return cum, nlt, before, nlt2, before2

        z = jnp.int32(0)
        total, chunk, beforec, chunk2, beforec2 = lax.fori_loop(
            0, nch, body, (z, z, z, z, z), unroll=8
        )
        chunk2 = jnp.minimum(chunk2, nch - 1)
        if remap:
            chunk = (chunk + nch // 2) & (nch - 1)
            chunk2 = (chunk2 + nch // 2) & (nch - 1)
        rr = r - beforec
        v = hist[pl.ds(base + chunk * 16, 16)]
        cum = plsc.cumsum(v)
        lt = cum < rr
        nlt = jnp.sum(lt.astype(jnp.int32))
        before_in = jnp.max(jnp.where(lt, cum, 0))
        bin_a = chunk * 16 + nlt
        before_a = beforec + before_in
        rr2 = rn - beforec2
        v2 = hist[pl.ds(base + chunk2 * 16, 16)]
        cum2 = plsc.cumsum(v2)
        nlt2f = jnp.sum((cum2 < rr2).astype(jnp.int32))
        bin_b = chunk2 * 16 + nlt2f
        return bin_a, before_a, bin_b, total

    base = wid * ROWS_PER_W
    bufs = [(row_a, in_a, out_a), (row_b, in_b, out_b)]
    in_handles = [pltpu.async_copy(x_hbm.at[base], row_a, in_a), None]
    out_handles = [None, None]

    for i in range(ROWS_PER_W):
        row_f, in_sem, out_sem = bufs[i % 2]
        in_handles[i % 2].wait()

        with jax.named_scope("zero1"):
            zero_hist(NBINS)

        with jax.named_scope("pass1"):

            @plsc.parallel_loop(0, CHUNKS, 1, unroll=4)
            def _(c, row_f=row_f):
                xv = row_f[pl.ds(c * 16, 16)]
                u = lax.bitcast_convert_type(xv, jnp.int32)
                kd = _desc_key(u)
                row_kd[pl.ds(c * 16, 16)] = kd
                bins = lax.shift_right_logical(kd, 21)
                plsc.addupdate_scatter(hist, [bins], ones16)

        if i + 1 < ROWS_PER_W:
            nxt_f, nxt_in, _ = bufs[(i + 1) % 2]
            if out_handles[(i + 1) % 2] is not None:
                out_handles[(i + 1) % 2].wait()
                out_handles[(i + 1) % 2] = None
            in_handles[(i + 1) % 2] = pltpu.async_copy(
                x_hbm.at[base + i + 1], nxt_f, nxt_in
            )

        with jax.named_scope("scan1"):
            r1 = jnp.int32(K_RANK)
            b1, before1, b1n, _tot1 = scan_hist2(0, NBINS, r1, remap=True)
            r2 = r1 - before1
            d1 = b1n != b1

        with jax.named_scope("pass2"):
            zero_hist(NBINS)

            @plsc.parallel_loop(0, CHUNKS, 1, unroll=4)
            def _(c):
                kd = row_kd[pl.ds(c * 16, 16)]
                m = lax.shift_right_logical(kd, 21) == b1
                bins = lax.shift_right_logical(kd, 10) & 0x7FF
                plsc.addupdate_scatter(hist, [bins], ones16, mask=m)

        with jax.named_scope("scan2"):
            b2, before2, b2n, _tot2 = scan_hist2(0, NBINS, r2)
            r3 = r2 - before2
            p2 = (b1 << 11) | b2
            p2n = (b1 << 11) | b2n
            same3 = p2n == p2
            p2n_sent = jnp.where(same3, jnp.int32(-1), p2n)

        with jax.named_scope("pass3"):
            zero_hist(NBINS)

            @plsc.parallel_loop(0, CHUNKS, 1, unroll=4)
            def _(c):
                kd = row_kd[pl.ds(c * 16, 16)]
                pfx = lax.shift_right_logical(kd, 10)
                m1 = pfx == p2
                m2 = pfx == p2n_sent
                bins = (kd & 0x3FF) + jnp.where(m2, 1024, 0)
                plsc.addupdate_scatter(hist, [bins], ones16, mask=m1 | m2)

        with jax.named_scope("scan3"):
            b3, _before3, b3n_same, _tot3 = scan_hist2(0, 1024, r3)
            b3n_diff, _bd = scan_hist(1024, 1024, jnp.int32(1))
            k1_kd = ((p2 << 10) | b3).astype(jnp.int32)
            k2_fast_kd = jnp.where(
                same3, (p2 << 10) | b3n_same, (p2n << 10) | b3n_diff
            ).astype(jnp.int32)

        with jax.named_scope("k2cond"):

            def k2_slow(k1_kd=k1_kd):
                maxs16 = jnp.full((16,), INT_MAX_I32, jnp.int32)

                @plsc.parallel_loop(0, CHUNKS, 1, unroll=4, carry=maxs16)
                def p4(c, mn):
                    kd = row_kd[pl.ds(c * 16, 16)]
                    return jnp.minimum(
                        mn, jnp.where(kd <= k1_kd, INT_MAX_I32, kd)
                    )

                return jnp.min(p4)

            k2_kd = lax.cond(d1, k2_slow, lambda k2=k2_fast_kd: k2)

        k1v = jnp.full((16,), k1_kd, jnp.int32)
        k2v = jnp.full((16,), k2_kd, jnp.int32)
        va = lax.bitcast_convert_type(_inv_desc_key(k1v), jnp.float32)
        vb = lax.bitcast_convert_type(_inv_desc_key(k2v), jnp.float32)
        t = (va + vb) * jnp.float32(0.5)
        t = jnp.where(t == 0.0, jnp.float32(0.0), t)
        t_kd = _desc_key(lax.bitcast_convert_type(t, jnp.int32))

        with jax.named_scope("maskp"):

            @plsc.parallel_loop(0, CHUNKS, 1, unroll=4)
            def _(c, row_f=row_f, t_kd=t_kd):
                kd = row_kd[pl.ds(c * 16, 16)]
                row_f[pl.ds(c * 16, 16)] = jnp.where(
                    kd < t_kd, jnp.float32(1.0), jnp.float32(0.0)
                )

        out_handles[i % 2] = pltpu.async_copy(
            row_f, out_hbm.at[base + i], out_sem
        )

    for h in out_handles:
        if h is not None:
            h.wait()


def kernel(x):
    return _kwta_sc(x)


# speedup vs baseline: 1.4874x; 1.0806x over previous
import functools

import jax
import jax.numpy as jnp
from jax import lax
from jax.experimental import pallas as pl
from jax.experimental.pallas import tpu as pltpu
from jax.experimental.pallas import tpu_sc as plsc

B = 128
N = 32768
K_RANK = 1639
NWORKERS = 32
ROWS_PER_W = B // NWORKERS
CHUNKS = N // 16
NBINS = 2048
INT_MIN_I32 = jnp.int32(-(2**31))
INT_MAX_I32 = jnp.int32(2**31 - 1)


def _desc_key(u):
    return u ^ (INT_MIN_I32 | ~(u >> 31))


def _inv_desc_key(kd):
    return jnp.where(kd >= 0, kd ^ INT_MIN_I32, ~kd)


_mesh = plsc.VectorSubcoreMesh(core_axis_name="c", subcore_axis_name="s")


@functools.partial(
    pl.kernel,
    out_type=jax.ShapeDtypeStruct((B, N), jnp.float32),
    mesh=_mesh,
    compiler_params=pltpu.CompilerParams(needs_layout_passes=False),
    scratch_types=[
        pltpu.VMEM((N,), jnp.float32),
        pltpu.VMEM((N,), jnp.float32),
        pltpu.VMEM((N,), jnp.int32),
        pltpu.VMEM((NBINS,), jnp.int32),
        pltpu.SMEM((NBINS // 16,), jnp.int32),
        pltpu.SemaphoreType.DMA,
        pltpu.SemaphoreType.DMA,
        pltpu.SemaphoreType.DMA,
        pltpu.SemaphoreType.DMA,
    ],
)
def _kwta_sc(
    x_hbm, out_hbm, row_a, row_b, row_kd, hist, sums, in_a, in_b, out_a, out_b
):
    wid = lax.axis_index("s") * 2 + lax.axis_index("c")
    zeros16 = jnp.zeros((16,), jnp.int32)
    ones16 = jnp.ones((16,), jnp.int32)

    def zero_hist(nbins):
        @plsc.parallel_loop(0, nbins // 16, 1, unroll=8)
        def _(c):
            hist[pl.ds(c * 16, 16)] = zeros16

    def scan_hist(base, nbins, r):
        nch = nbins // 16

        @plsc.parallel_loop(0, nch, 1, unroll=8)
        def _(c):
            sums[c] = jnp.sum(hist[pl.ds(base + c * 16, 16)])

        def body(c, carry):
            csum, nlt, before = carry
            cum = csum + sums[c]
            lt = cum < r
            nlt = nlt + lt.astype(jnp.int32)
            before = jnp.where(lt, cum, before)
            return cum, nlt, before

        z = jnp.int32(0)
        _, chunk, beforec = lax.fori_loop(0, nch, body, (z, z, z), unroll=8)
        rr = r - beforec
        v = hist[pl.ds(base + chunk * 16, 16)]
        cum = plsc.cumsum(v)
        lt = cum < rr
        nlt = jnp.sum(lt.astype(jnp.int32))
        before_in = jnp.max(jnp.where(lt, cum, 0))
        return chunk * 16 + nlt, beforec + before_in

    def scan_hist2(base, nbins, r, remap=False):
        nch = nbins // 16

        @plsc.parallel_loop(0, nch, 1, unroll=8)
        def _(c):
            sums[c] = jnp.sum(hist[pl.ds(base + c * 16, 16)])

        rn = r + 1

        def body(c, carry):
            csum, nlt, before, nlt2, before2 = carry
            rc = (c + nch // 2) & (nch - 1) if remap else c
            cum = csum + sums[rc]
            lt = cum < r
            nlt = nlt + lt.astype(jnp.int32)
            before = jnp.where(lt, cum, before)
            lt2 = cum < rn
            nlt2 = nlt2 + lt2.astype(jnp.int32)
            before2 = jnp.where(lt2, cum, before2)
            return cum, nlt, before, nlt2, before2

        z = jnp.int32(0)
        total, chunk, beforec, chunk2, beforec2 = lax.fori_loop(
            0, nch, body, (z, z, z, z, z), unroll=8
        )
        chunk2 = jnp.minimum(chunk2, nch - 1)
        if remap:
            chunk = (chunk + nch // 2) & (nch - 1)
            chunk2 = (chunk2 + nch // 2) & (nch - 1)
        rr = r - beforec
        v = hist[pl.ds(base + chunk * 16, 16)]
        cum = plsc.cumsum(v)
        lt = cum < rr
        nlt = jnp.sum(lt.astype(jnp.int32))
        before_in = jnp.max(jnp.where(lt, cum, 0))
        bin_a = chunk * 16 + nlt
        before_a = beforec + before_in
        rr2 = rn - beforec2
        v2 = hist[pl.ds(base + chunk2 * 16, 16)]
        cum2 = plsc.cumsum(v2)
        nlt2f = jnp.sum((cum2 < rr2).astype(jnp.int32))
        bin_b = chunk2 * 16 + nlt2f
        return bin_a, before_a, bin_b, total

    base = wid * ROWS_PER_W
    bufs = [(row_a, in_a, out_a), (row_b, in_b, out_b)]
    in_handles = [pltpu.async_copy(x_hbm.at[base], row_a, in_a), None]
    out_handles = [None, None]

    for i in range(ROWS_PER_W):
        row_f, in_sem, out_sem = bufs[i % 2]
        in_handles[i % 2].wait()

        with jax.named_scope("zero1"):
            zero_hist(NBINS)

        with jax.named_scope("pass1"):

            @plsc.parallel_loop(0, CHUNKS, 1, unroll=8)
            def _(c, row_f=row_f):
                xv = row_f[pl.ds(c * 16, 16)]
                u = lax.bitcast_convert_type(xv, jnp.int32)
                kd = _desc_key(u)
                row_kd[pl.ds(c * 16, 16)] = kd
                bins = lax.shift_right_logical(kd, 21)
                plsc.addupdate_scatter(hist, [bins], ones16)

        if i + 1 < ROWS_PER_W:
            nxt_f, nxt_in, _ = bufs[(i + 1) % 2]
            if out_handles[(i + 1) % 2] is not None:
                out_handles[(i + 1) % 2].wait()
                out_handles[(i + 1) % 2] = None
            in_handles[(i + 1) % 2] = pltpu.async_copy(
                x_hbm.at[base + i + 1], nxt_f, nxt_in
            )

        with jax.named_scope("scan1"):
            r1 = jnp.int32(K_RANK)
            b1, before1, b1n, _tot1 = scan_hist2(0, NBINS, r1, remap=True)
            r2 = r1 - before1
            d1 = b1n != b1

        with jax.named_scope("pass2"):
            zero_hist(NBINS)

            @plsc.parallel_loop(0, CHUNKS, 1, unroll=8)
            def _(c):
                kd = row_kd[pl.ds(c * 16, 16)]
                m = lax.shift_right_logical(kd, 21) == b1
                bins = lax.shift_right_logical(kd, 10) & 0x7FF
                plsc.addupdate_scatter(hist, [bins], ones16, mask=m)

        with jax.named_scope("scan2"):
            b2, before2, b2n, _tot2 = scan_hist2(0, NBINS, r2)
            r3 = r2 - before2
            p2 = (b1 << 11) | b2
            p2n = (b1 << 11) | b2n
            same3 = p2n == p2
            p2n_sent = jnp.where(same3, jnp.int32(-1), p2n)

        with jax.named_scope("pass3"):
            zero_hist(NBINS)

            @plsc.parallel_loop(0, CHUNKS, 1, unroll=8)
            def _(c):
                kd = row_kd[pl.ds(c * 16, 16)]
                pfx = lax.shift_right_logical(kd, 10)
                m1 = pfx == p2
                m2 = pfx == p2n_sent
                bins = (kd & 0x3FF) + jnp.where(m2, 1024, 0)
                plsc.addupdate_scatter(hist, [bins], ones16, mask=m1 | m2)

        with jax.named_scope("scan3"):
            b3, _before3, b3n_same, _tot3 = scan_hist2(0, 1024, r3)
            b3n_diff, _bd = scan_hist(1024, 1024, jnp.int32(1))
            k1_kd = ((p2 << 10) | b3).astype(jnp.int32)
            k2_fast_kd = jnp.where(
                same3, (p2 << 10) | b3n_same, (p2n << 10) | b3n_diff
            ).astype(jnp.int32)

        with jax.named_scope("k2cond"):

            def k2_slow(k1_kd=k1_kd):
                maxs16 = jnp.full((16,), INT_MAX_I32, jnp.int32)

                @plsc.parallel_loop(0, CHUNKS, 1, unroll=8, carry=maxs16)
                def p4(c, mn):
                    kd = row_kd[pl.ds(c * 16, 16)]
                    return jnp.minimum(
                        mn, jnp.where(kd <= k1_kd, INT_MAX_I32, kd)
                    )

                return jnp.min(p4)

            k2_kd = lax.cond(d1, k2_slow, lambda k2=k2_fast_kd: k2)

        k1v = jnp.full((16,), k1_kd, jnp.int32)
        k2v = jnp.full((16,), k2_kd, jnp.int32)
        va = lax.bitcast_convert_type(_inv_desc_key(k1v), jnp.float32)
        vb = lax.bitcast_convert_type(_inv_desc_key(k2v), jnp.float32)
        t = (va + vb) * jnp.float32(0.5)
        t = jnp.where(t == 0.0, jnp.float32(0.0), t)
        t_kd = _desc_key(lax.bitcast_convert_type(t, jnp.int32))

        with jax.named_scope("maskp"):

            @plsc.parallel_loop(0, CHUNKS, 1, unroll=8)
            def _(c, row_f=row_f, t_kd=t_kd):
                kd = row_kd[pl.ds(c * 16, 16)]
                row_f[pl.ds(c * 16, 16)] = jnp.where(
                    kd < t_kd, jnp.float32(1.0), jnp.float32(0.0)
                )

        out_handles[i % 2] = pltpu.async_copy(
            row_f, out_hbm.at[base + i], out_sem
        )

    for h in out_handles:
        if h is not None:
            h.wait()


def kernel(x):
    return _kwta_sc(x)
